# baseline (speedup 1.0000x reference)
import functools

import jax
import jax.numpy as jnp
from jax import lax
from jax.experimental import pallas as pl
from jax.experimental.pallas import tpu as pltpu
from jax.experimental.pallas import tpu_sc as plsc

N = 100000
NSUB = 16
NCORE = 2
NWORK = NSUB * NCORE
CSEG = 32
NRANGE = N // CSEG
BE = 512
GW = 128
E_RAW = 3200000
E_PAD = -(-(E_RAW + NRANGE * BE) // (GW * NWORK)) * (GW * NWORK)
NWIN = E_PAD // GW
NBLK = E_PAD // BE
BN_EPS = 1e-5

_vector_mesh = plsc.VectorSubcoreMesh(core_axis_name="c", subcore_axis_name="s")


@functools.lru_cache(maxsize=None)
def _make_gather(f):
    @functools.partial(
        pl.kernel,
        mesh=_vector_mesh,
        out_type=jax.ShapeDtypeStruct((E_PAD, f), jnp.float32),
        scratch_types=[
            pltpu.VMEM((GW,), jnp.int32),
            pltpu.VMEM((GW, f), jnp.float32),
            pltpu.SemaphoreType.DMA,
        ],
    )
    def gat(table_hbm, psrc_hbm, msg_hbm, isrc, rows, sem):
        core = lax.axis_index("c")
        sid = lax.axis_index("s")
        wid = sid * NCORE + core

        @pl.loop(0, NWIN // NWORK)
        def _(k):
            b = (k * NWORK + wid) * GW
            pltpu.sync_copy(psrc_hbm.at[pl.ds(b, GW)], isrc)
            pltpu.async_copy(table_hbm.at[isrc], rows, sem).wait()
            pltpu.sync_copy(rows, msg_hbm.at[pl.ds(b, GW)])

    return gat


def _segsum(msg, ids3d, rng):
    f = msg.shape[1]

    def body(rng_ref, ids_ref, msg_ref, out_ref):
        i = pl.program_id(0)
        ids = ids_ref[0, 0, :]
        onehot = (lax.broadcasted_iota(jnp.int32, (CSEG, BE), 0)
                  == ids[None, :]).astype(jnp.float32)
        partial = jnp.dot(onehot, msg_ref[...],
                          preferred_element_type=jnp.float32,
                    precision=lax.Precision.HIGHEST)
        prev = rng_ref[jnp.maximum(i - 1, 0)]
        first = jnp.logical_or(i == 0, rng_ref[i] != prev)

        @pl.when(first)
        def _():
            out_ref[...] = partial

        @pl.when(jnp.logical_not(first))
        def _():
            out_ref[...] += partial

    grid_spec = pltpu.PrefetchScalarGridSpec(
        num_scalar_prefetch=1,
        grid=(NBLK,),
        in_specs=[
            pl.BlockSpec((1, 1, BE), lambda i, rng: (i, 0, 0)),
            pl.BlockSpec((BE, f), lambda i, rng: (i, 0)),
        ],
        out_specs=pl.BlockSpec((CSEG, f), lambda i, rng: (rng[i], 0)),
    )
    return pl.pallas_call(
        body,
        grid_spec=grid_spec,
        out_shape=jax.ShapeDtypeStruct((N, f), jnp.float32),
    )(rng, ids3d, msg)


def _deg(ids3d, rng):
    def body(rng_ref, ids_ref, out_ref):
        i = pl.program_id(0)
        ids = ids_ref[0, 0, :]
        onehot = (lax.broadcasted_iota(jnp.int32, (CSEG, BE), 0)
                  == ids[None, :]).astype(jnp.float32)
        partial = jnp.broadcast_to(
            jnp.sum(onehot, axis=1, keepdims=True), (CSEG, 128))
        prev = rng_ref[jnp.maximum(i - 1, 0)]
        first = jnp.logical_or(i == 0, rng_ref[i] != prev)

        @pl.when(first)
        def _():
            out_ref[...] = partial

        @pl.when(jnp.logical_not(first))
        def _():
            out_ref[...] += partial

    grid_spec = pltpu.PrefetchScalarGridSpec(
        num_scalar_prefetch=1,
        grid=(NBLK,),
        in_specs=[pl.BlockSpec((1, 1, BE), lambda i, rng: (i, 0, 0))],
        out_specs=pl.BlockSpec((CSEG, 128), lambda i, rng: (rng[i], 0)),
    )
    return pl.pallas_call(
        body,
        grid_spec=grid_spec,
        out_shape=jax.ShapeDtypeStruct((N, 128), jnp.float32),
    )(rng, ids3d)


_R = 2000
_NG = N // _R


def _row_specs(*fs):
    return [pl.BlockSpec((_R, f), lambda i: (i, 0)) for f in fs]


def _full_spec(shape):
    nd = len(shape)
    return pl.BlockSpec(shape, lambda i, _nd=nd: (0,) * _nd)


def _stats_accumulate(i, z, st_ref):
    @pl.when(i == 0)
    def _():
        st_ref[...] = jnp.zeros(st_ref.shape, st_ref.dtype)

    st_ref[0:1, :] += jnp.sum(z, axis=0, keepdims=True)

    @pl.when(i == _NG - 1)
    def _():
        st_ref[0:1, :] = st_ref[0:1, :] / N


def _var(z, stm):
    f = z.shape[1]

    def body(z_ref, m_ref, st_ref):
        i = pl.program_id(0)
        d = z_ref[...] - m_ref[0:1, :]

        @pl.when(i == 0)
        def _():
            st_ref[...] = jnp.zeros(st_ref.shape, st_ref.dtype)

        st_ref[1:2, :] += jnp.sum(d * d, axis=0, keepdims=True)

        @pl.when(i == _NG - 1)
        def _():
            st_ref[0:1, :] = m_ref[0:1, :]
            st_ref[1:2, :] = lax.rsqrt(st_ref[1:2, :] / N + BN_EPS)

    return pl.pallas_call(
        body,
        grid=(_NG,),
        in_specs=[_row_specs(f)[0], _full_spec((8, f))],
        out_specs=_full_spec((8, f)),
        out_shape=jax.ShapeDtypeStruct((8, f), jnp.float32),
    )(z, stm)


def _affine_relu(x, st_ref, gb_ref):
    u = (x - st_ref[0:1, :]) * st_ref[1:2, :] * gb_ref[0:1, :] + gb_ref[1:2, :]
    return jnp.where(u > 0, u, 0.01 * u)


def _padw(x, fp):
    f = x.shape[1]
    if f == fp:
        return x
    return jnp.concatenate(
        [x, jnp.zeros((x.shape[0], fp - f), jnp.float32)], axis=1)


def _mw(f):
    return max(f, 128)


def _scale0(z1, dinv):
    f = z1.shape[1]
    fp = _mw(f)

    def body(z_ref, d_ref, g_ref):
        g_ref[...] = _padw(d_ref[...] * z_ref[...], fp)

    return pl.pallas_call(
        body,
        grid=(_NG,),
        in_specs=_row_specs(f, 1),
        out_specs=_row_specs(fp)[0],
        out_shape=jax.ShapeDtypeStruct((N, fp), jnp.float32),
    )(z1, dinv)


def _pre_scale(z, st, gb, dinv):
    f = z.shape[1]
    fp = _mw(f)

    def body(z_ref, st_ref, gb_ref, d_ref, g_ref):
        u = _affine_relu(z_ref[...], st_ref, gb_ref)
        g_ref[...] = _padw(d_ref[...] * u, fp)

    return pl.pallas_call(
        body,
        grid=(_NG,),
        in_specs=[_row_specs(f)[0], _full_spec((8, f)), _full_spec((8, f)),
                  _row_specs(1)[0]],
        out_specs=_row_specs(fp)[0],
        out_shape=jax.ShapeDtypeStruct((N, fp), jnp.float32),
    )(z, st, gb, dinv)


def _mm_pre(agg, g, dinv, w, brow):
    fin, fout = w.shape
    fp = agg.shape[1]

    def body(a_ref, g_ref, d_ref, w_ref, b_ref, z_ref, st_ref):
        i = pl.program_id(0)
        y = d_ref[...] * (a_ref[...] + g_ref[...])
        z = jnp.dot(y[:, :fin], w_ref[...],
                    preferred_element_type=jnp.float32,
                    precision=lax.Precision.HIGHEST)
        z = z + b_ref[0:1, :]
        z_ref[...] = z
        _stats_accumulate(i, z, st_ref)

    return pl.pallas_call(
        body,
        grid=(_NG,),
        in_specs=[_row_specs(fp)[0], _row_specs(fp)[0], _row_specs(1)[0],
                  _full_spec((fin, fout)), _full_spec((8, fout))],
        out_specs=[_row_specs(fout)[0], _full_spec((8, fout))],
        out_shape=[jax.ShapeDtypeStruct((N, fout), jnp.float32),
                   jax.ShapeDtypeStruct((8, fout), jnp.float32)],
    )(agg, g, dinv, w, brow)


def _mm_post(z, st, gb, w, dinv):
    fin, fout = w.shape
    fp = _mw(fout)

    def body(z_ref, st_ref, gb_ref, w_ref, d_ref, g_ref):
        u = _affine_relu(z_ref[...], st_ref, gb_ref)
        h = jnp.dot(u, w_ref[...], preferred_element_type=jnp.float32,
                    precision=lax.Precision.HIGHEST)
        g_ref[...] = _padw(d_ref[...] * h, fp)

    return pl.pallas_call(
        body,
        grid=(_NG,),
        in_specs=[_row_specs(fin)[0], _full_spec((8, fin)), _full_spec((8, fin)),
                  _full_spec((fin, fout)), _row_specs(1)[0]],
        out_specs=_row_specs(fp)[0],
        out_shape=jax.ShapeDtypeStruct((N, fp), jnp.float32),
    )(z, st, gb, w, dinv)


def _comb(agg, g, dinv, brow, fout):
    fp = g.shape[1]

    def body(a_ref, g_ref, d_ref, b_ref, z_ref, st_ref):
        i = pl.program_id(0)
        y = d_ref[...] * (a_ref[...] + g_ref[...])
        z = y[:, :fout] + b_ref[0:1, :]
        z_ref[...] = z
        _stats_accumulate(i, z, st_ref)

    return pl.pallas_call(
        body,
        grid=(_NG,),
        in_specs=[_row_specs(fp)[0], _row_specs(fp)[0], _row_specs(1)[0],
                  _full_spec((8, fout))],
        out_specs=[_row_specs(fout)[0], _full_spec((8, fout))],
        out_shape=[jax.ShapeDtypeStruct((N, fout), jnp.float32),
                   jax.ShapeDtypeStruct((8, fout), jnp.float32)],
    )(agg, g, dinv, brow)


def _dinv_from_deg(deg128):
    def body(d_ref, o_ref):
        deg = d_ref[:, 0:1] + 1.0
        o_ref[...] = lax.rsqrt(jnp.maximum(deg, 1.0))

    return pl.pallas_call(
        body,
        grid=(_NG,),
        in_specs=_row_specs(128),
        out_specs=_row_specs(1)[0],
        out_shape=jax.ShapeDtypeStruct((N, 1), jnp.float32),
    )(deg128)


def _final(z, st, gb, w1, b1row, w2, b2row, x_pos):
    fin = z.shape[1]
    f1 = w1.shape[1]
    f2 = w2.shape[1]

    def body(z_ref, st_ref, gb_ref, w1_ref, b1_ref, w2_ref, b2_ref, xp_ref,
             o_ref):
        u = _affine_relu(z_ref[...], st_ref, gb_ref)
        t = jnp.dot(u, w1_ref[...], preferred_element_type=jnp.float32,
                    precision=lax.Precision.HIGHEST)
        t = t + b1_ref[0:1, :]
        t = jnp.where(t > 0, t, 0.01 * t)
        o = jnp.dot(t, w2_ref[...], preferred_element_type=jnp.float32,
                    precision=lax.Precision.HIGHEST)
        o_ref[...] = o + b2_ref[0:1, :] + xp_ref[...]

    return pl.pallas_call(
        body,
        grid=(_NG,),
        in_specs=[_row_specs(fin)[0], _full_spec((8, fin)), _full_spec((8, fin)),
                  _full_spec((fin, f1)), _full_spec((8, f1)),
                  _full_spec((f1, f2)), _full_spec((8, f2)),
                  _row_specs(f2)[0]],
        out_specs=_row_specs(f2)[0],
        out_shape=jax.ShapeDtypeStruct((N, f2), jnp.float32),
    )(z, st, gb, w1, b1row, w2, b2row, x_pos)


def _row8(v):
    f = v.shape[0]
    return jnp.zeros((8, f), jnp.float32).at[0, :].set(v)


def _rows8(g, b):
    f = g.shape[0]
    out = jnp.zeros((8, f), jnp.float32)
    return out.at[0, :].set(g).at[1, :].set(b)


def kernel(z1, x_pos, params, edge_index):
    n, f0 = z1.shape
    e = edge_index.shape[1]

    src = edge_index[0].astype(jnp.int32)
    dst = edge_index[1].astype(jnp.int32)
    order = jnp.argsort(dst)
    dst_s = dst[order]
    src_s = src[order]
    range_of = dst_s // CSEG
    bounds = jnp.searchsorted(
        dst_s, jnp.arange(0, N + 1, CSEG, dtype=jnp.int32)).astype(jnp.int32)
    cnt = bounds[1:] - bounds[:-1]
    pcnt = jnp.maximum((cnt + BE - 1) // BE, 1) * BE
    poff = jnp.concatenate(
        [jnp.zeros((1,), jnp.int32), jnp.cumsum(pcnt).astype(jnp.int32)])
    pos = poff[range_of] + (jnp.arange(e, dtype=jnp.int32) - bounds[range_of])
    psrc = jnp.zeros((E_PAD,), jnp.int32).at[pos].set(src_s)
    pldst = jnp.full((E_PAD,), CSEG, jnp.int32).at[pos].set(
        dst_s - range_of * CSEG)
    ids3d = pldst.reshape(NBLK, 1, BE)
    rng = (jnp.searchsorted(poff, jnp.arange(NBLK, dtype=jnp.int32) * BE,
                            side="right").astype(jnp.int32) - 1)
    rng = jnp.clip(rng, 0, NRANGE - 1)

    deg128 = _deg(ids3d, rng)
    dinv = _dinv_from_deg(deg128)

    pre_layer = [fi <= fo for (wmat, _, _, _) in params[:12]
                 for (fi, fo) in [wmat.shape]]

    z = None
    st = None
    for i in range(12):
        wmat, b, gamma, beta = params[i]
        brow = _row8(b)
        if pre_layer[i]:
            if i == 0:
                g = _scale0(z1, dinv)
            else:
                g = _pre_scale(z, st, _rows8(*params[i - 1][2:4]), dinv)
            msg = _make_gather(g.shape[1])(g, psrc)
            agg = _segsum(msg, ids3d, rng)
            z, st = _mm_pre(agg, g, dinv, wmat, brow)
            st = _var(z, st)
        else:
            g = _mm_post(z, st, _rows8(*params[i - 1][2:4]), wmat, dinv)
            msg = _make_gather(g.shape[1])(g, psrc)
            agg = _segsum(msg, ids3d, rng)
            z, st = _comb(agg, g, dinv, brow, wmat.shape[1])
            st = _var(z, st)

    w1, b1 = params[12]
    w2, b2 = params[13]
    return _final(z, st, _rows8(*params[11][2:4]), w1, _row8(b1),
                  w2, _row8(b2), x_pos)


# speedup vs baseline: 9.4035x; 9.4035x over previous
import functools

import jax
import jax.numpy as jnp
from jax import lax
from jax.experimental import pallas as pl
from jax.experimental.pallas import tpu as pltpu
from jax.experimental.pallas import tpu_sc as plsc

N = 100000
NSUB = 16
NCORE = 2
NWORK = NSUB * NCORE
CSEG = 32
NRANGE = N // CSEG
BE = 512
GW = 128
E_RAW = 3200000
E_PAD = -(-(E_RAW + NRANGE * BE) // (GW * NWORK)) * (GW * NWORK)
NWIN = E_PAD // GW
NBLK = E_PAD // BE
BN_EPS = 1e-5

_vector_mesh = plsc.VectorSubcoreMesh(core_axis_name="c", subcore_axis_name="s")


@functools.lru_cache(maxsize=None)
def _make_gather(f):
    gw = GW

    @functools.partial(
        pl.kernel,
        mesh=_vector_mesh,
        out_type=jax.ShapeDtypeStruct((E_PAD, f), jnp.float32),
        scratch_types=[],
    )
    def gat(table_hbm, psrc_hbm, msg_hbm):
        def body(i_vmem, o_vmem):
            pltpu.sync_copy(table_hbm.at[i_vmem.at[0]], o_vmem)

        pltpu.emit_pipeline(
            body,
            grid=(E_PAD // gw,),
            in_specs=[pl.BlockSpec((1, gw), lambda i: (0, i))],
            out_specs=[pl.BlockSpec((gw, f), lambda i: (i, 0))],
            core_axis_name=("c", "s"),
            dimension_semantics=(pltpu.PARALLEL,),
        )(psrc_hbm, msg_hbm)

    return gat


def _gather_segsum(g, psrc, ids3d, rng):
    f = g.shape[1]
    if f <= 256:
        msg = _make_gather(f)(g, psrc)
        return _segsum(msg, ids3d, rng)
    halves = []
    for lo in range(0, f, 256):
        msg = _make_gather(256)(g[:, lo:lo + 256], psrc)
        halves.append(_segsum(msg, ids3d, rng))
    return jnp.concatenate(halves, axis=1)


def _segsum(msg, ids3d, rng):
    f = msg.shape[1]

    def body(rng_ref, ids_ref, msg_ref, out_ref):
        i = pl.program_id(0)
        ids = ids_ref[0, 0, :]
        onehot = (lax.broadcasted_iota(jnp.int32, (CSEG, BE), 0)
                  == ids[None, :]).astype(jnp.float32)
        partial = jnp.dot(onehot, msg_ref[...],
                          preferred_element_type=jnp.float32,
                    precision=lax.Precision.HIGHEST)
        prev = rng_ref[jnp.maximum(i - 1, 0)]
        first = jnp.logical_or(i == 0, rng_ref[i] != prev)

        @pl.when(first)
        def _():
            out_ref[...] = partial

        @pl.when(jnp.logical_not(first))
        def _():
            out_ref[...] += partial

    grid_spec = pltpu.PrefetchScalarGridSpec(
        num_scalar_prefetch=1,
        grid=(NBLK,),
        in_specs=[
            pl.BlockSpec((1, 1, BE), lambda i, rng: (i, 0, 0)),
            pl.BlockSpec((BE, f), lambda i, rng: (i, 0)),
        ],
        out_specs=pl.BlockSpec((CSEG, f), lambda i, rng: (rng[i], 0)),
    )
    return pl.pallas_call(
        body,
        grid_spec=grid_spec,
        out_shape=jax.ShapeDtypeStruct((N, f), jnp.float32),
    )(rng, ids3d, msg)


def _deg(ids3d, rng):
    def body(rng_ref, ids_ref, out_ref):
        i = pl.program_id(0)
        ids = ids_ref[0, 0, :]
        onehot = (lax.broadcasted_iota(jnp.int32, (CSEG, BE), 0)
                  == ids[None, :]).astype(jnp.float32)
        partial = jnp.broadcast_to(
            jnp.sum(onehot, axis=1, keepdims=True), (CSEG, 128))
        prev = rng_ref[jnp.maximum(i - 1, 0)]
        first = jnp.logical_or(i == 0, rng_ref[i] != prev)

        @pl.when(first)
        def _():
            out_ref[...] = partial

        @pl.when(jnp.logical_not(first))
        def _():
            out_ref[...] += partial

    grid_spec = pltpu.PrefetchScalarGridSpec(
        num_scalar_prefetch=1,
        grid=(NBLK,),
        in_specs=[pl.BlockSpec((1, 1, BE), lambda i, rng: (i, 0, 0))],
        out_specs=pl.BlockSpec((CSEG, 128), lambda i, rng: (rng[i], 0)),
    )
    return pl.pallas_call(
        body,
        grid_spec=grid_spec,
        out_shape=jax.ShapeDtypeStruct((N, 128), jnp.float32),
    )(rng, ids3d)


_R = 2000
_NG = N // _R


def _row_specs(*fs):
    return [pl.BlockSpec((_R, f), lambda i: (i, 0)) for f in fs]


def _full_spec(shape):
    nd = len(shape)
    return pl.BlockSpec(shape, lambda i, _nd=nd: (0,) * _nd)


def _stats_accumulate(i, z, st_ref):
    @pl.when(i == 0)
    def _():
        st_ref[...] = jnp.zeros(st_ref.shape, st_ref.dtype)

    st_ref[0:1, :] += jnp.sum(z, axis=0, keepdims=True)

    @pl.when(i == _NG - 1)
    def _():
        st_ref[0:1, :] = st_ref[0:1, :] / N


def _var(z, stm):
    f = z.shape[1]

    def body(z_ref, m_ref, st_ref):
        i = pl.program_id(0)
        d = z_ref[...] - m_ref[0:1, :]

        @pl.when(i == 0)
        def _():
            st_ref[...] = jnp.zeros(st_ref.shape, st_ref.dtype)

        st_ref[1:2, :] += jnp.sum(d * d, axis=0, keepdims=True)

        @pl.when(i == _NG - 1)
        def _():
            st_ref[0:1, :] = m_ref[0:1, :]
            st_ref[1:2, :] = lax.rsqrt(st_ref[1:2, :] / N + BN_EPS)

    return pl.pallas_call(
        body,
        grid=(_NG,),
        in_specs=[_row_specs(f)[0], _full_spec((8, f))],
        out_specs=_full_spec((8, f)),
        out_shape=jax.ShapeDtypeStruct((8, f), jnp.float32),
    )(z, stm)


def _affine_relu(x, st_ref, gb_ref):
    u = (x - st_ref[0:1, :]) * st_ref[1:2, :] * gb_ref[0:1, :] + gb_ref[1:2, :]
    return jnp.where(u > 0, u, 0.01 * u)


def _padw(x, fp):
    f = x.shape[1]
    if f == fp:
        return x
    return jnp.concatenate(
        [x, jnp.zeros((x.shape[0], fp - f), jnp.float32)], axis=1)


def _mw(f):
    return max(f, 128)


def _scale0(z1, dinv):
    f = z1.shape[1]
    fp = _mw(f)

    def body(z_ref, d_ref, g_ref):
        g_ref[...] = _padw(d_ref[...] * z_ref[...], fp)

    return pl.pallas_call(
        body,
        grid=(_NG,),
        in_specs=_row_specs(f, 1),
        out_specs=_row_specs(fp)[0],
        out_shape=jax.ShapeDtypeStruct((N, fp), jnp.float32),
    )(z1, dinv)


def _pre_scale(z, st, gb, dinv):
    f = z.shape[1]
    fp = _mw(f)

    def body(z_ref, st_ref, gb_ref, d_ref, g_ref):
        u = _affine_relu(z_ref[...], st_ref, gb_ref)
        g_ref[...] = _padw(d_ref[...] * u, fp)

    return pl.pallas_call(
        body,
        grid=(_NG,),
        in_specs=[_row_specs(f)[0], _full_spec((8, f)), _full_spec((8, f)),
                  _row_specs(1)[0]],
        out_specs=_row_specs(fp)[0],
        out_shape=jax.ShapeDtypeStruct((N, fp), jnp.float32),
    )(z, st, gb, dinv)


def _mm_pre(agg, g, dinv, w, brow):
    fin, fout = w.shape
    fp = agg.shape[1]

    def body(a_ref, g_ref, d_ref, w_ref, b_ref, z_ref, st_ref):
        i = pl.program_id(0)
        y = d_ref[...] * (a_ref[...] + g_ref[...])
        z = jnp.dot(y[:, :fin], w_ref[...],
                    preferred_element_type=jnp.float32,
                    precision=lax.Precision.HIGHEST)
        z = z + b_ref[0:1, :]
        z_ref[...] = z
        _stats_accumulate(i, z, st_ref)

    return pl.pallas_call(
        body,
        grid=(_NG,),
        in_specs=[_row_specs(fp)[0], _row_specs(fp)[0], _row_specs(1)[0],
                  _full_spec((fin, fout)), _full_spec((8, fout))],
        out_specs=[_row_specs(fout)[0], _full_spec((8, fout))],
        out_shape=[jax.ShapeDtypeStruct((N, fout), jnp.float32),
                   jax.ShapeDtypeStruct((8, fout), jnp.float32)],
    )(agg, g, dinv, w, brow)


def _mm_post(z, st, gb, w, dinv):
    fin, fout = w.shape
    fp = _mw(fout)

    def body(z_ref, st_ref, gb_ref, w_ref, d_ref, g_ref):
        u = _affine_relu(z_ref[...], st_ref, gb_ref)
        h = jnp.dot(u, w_ref[...], preferred_element_type=jnp.float32,
                    precision=lax.Precision.HIGHEST)
        g_ref[...] = _padw(d_ref[...] * h, fp)

    return pl.pallas_call(
        body,
        grid=(_NG,),
        in_specs=[_row_specs(fin)[0], _full_spec((8, fin)), _full_spec((8, fin)),
                  _full_spec((fin, fout)), _row_specs(1)[0]],
        out_specs=_row_specs(fp)[0],
        out_shape=jax.ShapeDtypeStruct((N, fp), jnp.float32),
    )(z, st, gb, w, dinv)


def _comb(agg, g, dinv, brow, fout):
    fp = g.shape[1]

    def body(a_ref, g_ref, d_ref, b_ref, z_ref, st_ref):
        i = pl.program_id(0)
        y = d_ref[...] * (a_ref[...] + g_ref[...])
        z = y[:, :fout] + b_ref[0:1, :]
        z_ref[...] = z
        _stats_accumulate(i, z, st_ref)

    return pl.pallas_call(
        body,
        grid=(_NG,),
        in_specs=[_row_specs(fp)[0], _row_specs(fp)[0], _row_specs(1)[0],
                  _full_spec((8, fout))],
        out_specs=[_row_specs(fout)[0], _full_spec((8, fout))],
        out_shape=[jax.ShapeDtypeStruct((N, fout), jnp.float32),
                   jax.ShapeDtypeStruct((8, fout), jnp.float32)],
    )(agg, g, dinv, brow)


def _dinv_from_deg(deg128):
    def body(d_ref, o_ref):
        deg = d_ref[:, 0:1] + 1.0
        o_ref[...] = lax.rsqrt(jnp.maximum(deg, 1.0))

    return pl.pallas_call(
        body,
        grid=(_NG,),
        in_specs=_row_specs(128),
        out_specs=_row_specs(1)[0],
        out_shape=jax.ShapeDtypeStruct((N, 1), jnp.float32),
    )(deg128)


def _final(z, st, gb, w1, b1row, w2, b2row, x_pos):
    fin = z.shape[1]
    f1 = w1.shape[1]
    f2 = w2.shape[1]

    def body(z_ref, st_ref, gb_ref, w1_ref, b1_ref, w2_ref, b2_ref, xp_ref,
             o_ref):
        u = _affine_relu(z_ref[...], st_ref, gb_ref)
        t = jnp.dot(u, w1_ref[...], preferred_element_type=jnp.float32,
                    precision=lax.Precision.HIGHEST)
        t = t + b1_ref[0:1, :]
        t = jnp.where(t > 0, t, 0.01 * t)
        o = jnp.dot(t, w2_ref[...], preferred_element_type=jnp.float32,
                    precision=lax.Precision.HIGHEST)
        o_ref[...] = o + b2_ref[0:1, :] + xp_ref[...]

    return pl.pallas_call(
        body,
        grid=(_NG,),
        in_specs=[_row_specs(fin)[0], _full_spec((8, fin)), _full_spec((8, fin)),
                  _full_spec((fin, f1)), _full_spec((8, f1)),
                  _full_spec((f1, f2)), _full_spec((8, f2)),
                  _row_specs(f2)[0]],
        out_specs=_row_specs(f2)[0],
        out_shape=jax.ShapeDtypeStruct((N, f2), jnp.float32),
    )(z, st, gb, w1, b1row, w2, b2row, x_pos)


def _row8(v):
    f = v.shape[0]
    return jnp.zeros((8, f), jnp.float32).at[0, :].set(v)


def _rows8(g, b):
    f = g.shape[0]
    out = jnp.zeros((8, f), jnp.float32)
    return out.at[0, :].set(g).at[1, :].set(b)


def kernel(z1, x_pos, params, edge_index):
    n, f0 = z1.shape
    e = edge_index.shape[1]

    src = edge_index[0].astype(jnp.int32)
    dst = edge_index[1].astype(jnp.int32)
    order = jnp.argsort(dst)
    dst_s = dst[order]
    src_s = src[order]
    range_of = dst_s // CSEG
    bounds = jnp.searchsorted(
        dst_s, jnp.arange(0, N + 1, CSEG, dtype=jnp.int32)).astype(jnp.int32)
    cnt = bounds[1:] - bounds[:-1]
    pcnt = jnp.maximum((cnt + BE - 1) // BE, 1) * BE
    poff = jnp.concatenate(
        [jnp.zeros((1,), jnp.int32), jnp.cumsum(pcnt).astype(jnp.int32)])
    pos = poff[range_of] + (jnp.arange(e, dtype=jnp.int32) - bounds[range_of])
    psrc = jnp.zeros((E_PAD,), jnp.int32).at[pos].set(src_s).reshape(1, E_PAD)
    pldst = jnp.full((E_PAD,), CSEG, jnp.int32).at[pos].set(
        dst_s - range_of * CSEG)
    ids3d = pldst.reshape(NBLK, 1, BE)
    rng = (jnp.searchsorted(poff, jnp.arange(NBLK, dtype=jnp.int32) * BE,
                            side="right").astype(jnp.int32) - 1)
    rng = jnp.clip(rng, 0, NRANGE - 1)

    deg128 = _deg(ids3d, rng)
    dinv = _dinv_from_deg(deg128)

    pre_layer = [fi <= fo for (wmat, _, _, _) in params[:12]
                 for (fi, fo) in [wmat.shape]]

    z = None
    st = None
    for i in range(12):
        wmat, b, gamma, beta = params[i]
        brow = _row8(b)
        if pre_layer[i]:
            if i == 0:
                g = _scale0(z1, dinv)
            else:
                g = _pre_scale(z, st, _rows8(*params[i - 1][2:4]), dinv)
            agg = _gather_segsum(g, psrc, ids3d, rng)
            z, st = _mm_pre(agg, g, dinv, wmat, brow)
            st = _var(z, st)
        else:
            g = _mm_post(z, st, _rows8(*params[i - 1][2:4]), wmat, dinv)
            agg = _gather_segsum(g, psrc, ids3d, rng)
            z, st = _comb(agg, g, dinv, brow, wmat.shape[1])
            st = _var(z, st)

    w1, b1 = params[12]
    w2, b2 = params[13]
    return _final(z, st, _rows8(*params[11][2:4]), w1, _row8(b1),
                  w2, _row8(b2), x_pos)
